# baseline (speedup 1.0000x reference)
import jax
import jax.numpy as jnp
from jax.experimental import pallas as pl
from jax.experimental.pallas import tpu as pltpu

_NUM_LEVELS = 4
_K = 8192
_D = 256
_BETA = 0.25
_N_TOK = 2304
_T = 256
_NB = _N_TOK // _T


def _rvq_body(z_ref, cbt_ref, cb_ref,
              zq_out, idx_out, loss_out, cbl_out, coml_out, perp_out,
              res_s, qsum_s, counts_s, sse_s):
    l = pl.program_id(0)
    nb = pl.program_id(1)
    base = nb * _T

    @pl.when(jnp.logical_and(l == 0, nb == 0))
    def _init_global():
        counts_s[...] = jnp.zeros((1, _K), jnp.float32)
        sse_s[...] = jnp.zeros((1, 1), jnp.float32)

    @pl.when(l == 0)
    def _init_block():
        res_s[pl.ds(base, _T), :] = z_ref[...]
        qsum_s[pl.ds(base, _T), :] = jnp.zeros((_T, _D), jnp.float32)

    res = res_s[pl.ds(base, _T), :]
    cbt = cbt_ref[0]
    cb = cb_ref[0]

    c_sq = jnp.sum(cbt * cbt, axis=0, keepdims=True)
    dot = jnp.dot(res, cbt)
    d = c_sq - 2.0 * dot
    mval = jnp.min(d, axis=1, keepdims=True)
    lane = jax.lax.broadcasted_iota(jnp.int32, (_T, _K), 1)
    idx = jnp.min(jnp.where(d == mval, lane, _K), axis=1, keepdims=True)

    onehot = (lane == idx).astype(jnp.float32)
    zq = jax.lax.dot_general(onehot, cb, (((1,), (0,)), ((), ())),
                             precision=jax.lax.Precision.HIGHEST)

    counts_s[...] += jnp.sum(onehot, axis=0, keepdims=True)
    sse_s[...] += jnp.sum((zq - res) ** 2, keepdims=True).reshape(1, 1)

    eps = 1e-6
    rn = jnp.sqrt(jnp.sum(res * res, axis=1, keepdims=True))
    u = res / jnp.maximum(rn, eps)
    qn = jnp.sqrt(jnp.sum(zq * zq, axis=1, keepdims=True))
    q = zq / jnp.maximum(qn, eps)
    wv = u + q
    wn = jnp.sqrt(jnp.sum(wv * wv, axis=1, keepdims=True))
    w = wv / jnp.maximum(wn, eps)
    xw = jnp.sum(res * w, axis=1, keepdims=True)
    xu = jnp.sum(res * u, axis=1, keepdims=True)
    rot = res - 2.0 * xw * w + 2.0 * xu * q
    res_s[pl.ds(base, _T), :] = res - rot

    newq = qsum_s[pl.ds(base, _T), :] + zq
    qsum_s[pl.ds(base, _T), :] = newq
    zq_out[...] = newq
    idx_out[0, pl.ds(base, _T), :] = idx

    @pl.when(jnp.logical_and(l == _NUM_LEVELS - 1, nb == _NB - 1))
    def _finalize():
        n_el = jnp.float32(_N_TOK * _D)
        cbl = sse_s[...] / n_el
        probs = counts_s[...] / jnp.float32(_N_TOK * _NUM_LEVELS)
        safe = jnp.where(probs > 0, probs, 1.0)
        ent = -jnp.sum(jnp.where(probs > 0, probs * jnp.log(safe), 0.0),
                       keepdims=True).reshape(1, 1)
        loss_out[...] = cbl * (1.0 + _BETA)
        cbl_out[...] = cbl
        coml_out[...] = cbl
        perp_out[...] = jnp.exp(ent)


def kernel(z, codebooks):
    zf = z.reshape(_N_TOK, _D)
    cbt = codebooks.transpose(0, 2, 1)

    out_shapes = (
        jax.ShapeDtypeStruct((_N_TOK, _D), jnp.float32),
        jax.ShapeDtypeStruct((_NUM_LEVELS, _N_TOK, 1), jnp.int32),
        jax.ShapeDtypeStruct((1, 1), jnp.float32),
        jax.ShapeDtypeStruct((1, 1), jnp.float32),
        jax.ShapeDtypeStruct((1, 1), jnp.float32),
        jax.ShapeDtypeStruct((1, 1), jnp.float32),
    )
    grid = (_NUM_LEVELS, _NB)
    scalar_spec = pl.BlockSpec((1, 1), lambda l, nb: (0, 0))
    zq_flat, idx3, loss, cbl, coml, perp = pl.pallas_call(
        _rvq_body,
        grid=grid,
        in_specs=[
            pl.BlockSpec((_T, _D), lambda l, nb: (nb, 0)),
            pl.BlockSpec((1, _D, _K), lambda l, nb: (l, 0, 0)),
            pl.BlockSpec((1, _K, _D), lambda l, nb: (l, 0, 0)),
        ],
        out_specs=(
            pl.BlockSpec((_T, _D), lambda l, nb: (nb, 0)),
            pl.BlockSpec((1, _N_TOK, 1), lambda l, nb: (l, 0, 0)),
            scalar_spec, scalar_spec, scalar_spec, scalar_spec,
        ),
        out_shape=out_shapes,
        scratch_shapes=[
            pltpu.VMEM((_N_TOK, _D), jnp.float32),
            pltpu.VMEM((_N_TOK, _D), jnp.float32),
            pltpu.VMEM((1, _K), jnp.float32),
            pltpu.VMEM((1, 1), jnp.float32),
        ],
        compiler_params=pltpu.CompilerParams(
            dimension_semantics=("arbitrary", "arbitrary"),
        ),
    )(zf, cbt, codebooks)

    z_q = zq_flat.reshape(z.shape)
    indices = idx3[:, :, 0].T.reshape(z.shape[0], z.shape[1], _NUM_LEVELS)
    return (z_q, indices, loss.reshape(()), cbl.reshape(()),
            coml.reshape(()), perp.reshape(()))


# speedup vs baseline: 1.1746x; 1.1746x over previous
import jax
import jax.numpy as jnp
from jax.experimental import pallas as pl
from jax.experimental.pallas import tpu as pltpu

_NUM_LEVELS = 4
_K = 8192
_D = 256
_BETA = 0.25
_N_TOK = 2304
_T = 256
_NB = _N_TOK // _T


def _rvq_body(z_ref, cbt_ref, cb_ref, cb_lo_ref, cb_lo2_ref,
              zq_out, idx_out, loss_out, cbl_out, coml_out, perp_out,
              res_s, qsum_s, counts_s, sse_s, csq_s):
    l = pl.program_id(0)
    nb = pl.program_id(1)
    base = nb * _T

    @pl.when(jnp.logical_and(l == 0, nb == 0))
    def _init_global():
        counts_s[...] = jnp.zeros((1, _K), jnp.float32)
        sse_s[...] = jnp.zeros((1, 1), jnp.float32)

    @pl.when(l == 0)
    def _init_block():
        res_s[pl.ds(base, _T), :] = z_ref[...]
        qsum_s[pl.ds(base, _T), :] = jnp.zeros((_T, _D), jnp.float32)

    res = res_s[pl.ds(base, _T), :]
    cbt = cbt_ref[0]

    @pl.when(nb == 0)
    def _csq():
        csq_s[...] = jnp.sum(cbt * cbt, axis=0, keepdims=True)

    c_sq = csq_s[...]
    dot = jnp.dot(res, cbt)
    d = c_sq - 2.0 * dot
    mval = jnp.min(d, axis=1, keepdims=True)
    lane = jax.lax.broadcasted_iota(jnp.int32, (_T, _K), 1)
    idx = jnp.min(jnp.where(d == mval, lane, _K), axis=1, keepdims=True)

    onehot = (lane == idx).astype(jnp.bfloat16)
    cb_hi = cb_ref[0]
    cb_lo = cb_lo_ref[0]
    cb_lo2 = cb_lo2_ref[0]
    dn = (((1,), (0,)), ((), ()))
    zq = jax.lax.dot_general(onehot, cb_hi, dn,
                             preferred_element_type=jnp.float32)
    zq += jax.lax.dot_general(onehot, cb_lo, dn,
                              preferred_element_type=jnp.float32)
    zq += jax.lax.dot_general(onehot, cb_lo2, dn,
                              preferred_element_type=jnp.float32)

    counts_s[...] += jnp.sum(onehot.astype(jnp.float32), axis=0,
                             keepdims=True)
    sse_s[...] += jnp.sum((zq - res) ** 2, keepdims=True).reshape(1, 1)

    eps = 1e-6
    rn = jnp.sqrt(jnp.sum(res * res, axis=1, keepdims=True))
    u = res / jnp.maximum(rn, eps)
    qn = jnp.sqrt(jnp.sum(zq * zq, axis=1, keepdims=True))
    q = zq / jnp.maximum(qn, eps)
    wv = u + q
    wn = jnp.sqrt(jnp.sum(wv * wv, axis=1, keepdims=True))
    w = wv / jnp.maximum(wn, eps)
    xw = jnp.sum(res * w, axis=1, keepdims=True)
    xu = jnp.sum(res * u, axis=1, keepdims=True)
    rot = res - 2.0 * xw * w + 2.0 * xu * q
    res_s[pl.ds(base, _T), :] = res - rot

    newq = qsum_s[pl.ds(base, _T), :] + zq
    qsum_s[pl.ds(base, _T), :] = newq
    zq_out[...] = newq
    idx_out[0, pl.ds(base, _T), :] = idx

    @pl.when(jnp.logical_and(l == _NUM_LEVELS - 1, nb == _NB - 1))
    def _finalize():
        n_el = jnp.float32(_N_TOK * _D)
        cbl = sse_s[...] / n_el
        probs = counts_s[...] / jnp.float32(_N_TOK * _NUM_LEVELS)
        safe = jnp.where(probs > 0, probs, 1.0)
        ent = -jnp.sum(jnp.where(probs > 0, probs * jnp.log(safe), 0.0),
                       keepdims=True).reshape(1, 1)
        loss_out[...] = cbl * (1.0 + _BETA)
        cbl_out[...] = cbl
        coml_out[...] = cbl
        perp_out[...] = jnp.exp(ent)


def kernel(z, codebooks):
    zf = z.reshape(_N_TOK, _D)
    cbt = codebooks.transpose(0, 2, 1)
    mask = jnp.uint32(0xFFFF0000)
    u0 = jax.lax.bitcast_convert_type(codebooks, jnp.uint32)
    hi_f = jax.lax.bitcast_convert_type(u0 & mask, jnp.float32)
    r1 = codebooks - hi_f
    u1 = jax.lax.bitcast_convert_type(r1, jnp.uint32)
    lo_f = jax.lax.bitcast_convert_type(u1 & mask, jnp.float32)
    r2 = r1 - lo_f
    cb_hi = hi_f.astype(jnp.bfloat16)
    cb_lo = lo_f.astype(jnp.bfloat16)
    cb_lo2 = r2.astype(jnp.bfloat16)

    out_shapes = (
        jax.ShapeDtypeStruct((_N_TOK, _D), jnp.float32),
        jax.ShapeDtypeStruct((_NUM_LEVELS, _N_TOK, 1), jnp.int32),
        jax.ShapeDtypeStruct((1, 1), jnp.float32),
        jax.ShapeDtypeStruct((1, 1), jnp.float32),
        jax.ShapeDtypeStruct((1, 1), jnp.float32),
        jax.ShapeDtypeStruct((1, 1), jnp.float32),
    )
    grid = (_NUM_LEVELS, _NB)
    scalar_spec = pl.BlockSpec((1, 1), lambda l, nb: (0, 0))
    zq_flat, idx3, loss, cbl, coml, perp = pl.pallas_call(
        _rvq_body,
        grid=grid,
        in_specs=[
            pl.BlockSpec((_T, _D), lambda l, nb: (nb, 0)),
            pl.BlockSpec((1, _D, _K), lambda l, nb: (l, 0, 0)),
            pl.BlockSpec((1, _K, _D), lambda l, nb: (l, 0, 0)),
            pl.BlockSpec((1, _K, _D), lambda l, nb: (l, 0, 0)),
            pl.BlockSpec((1, _K, _D), lambda l, nb: (l, 0, 0)),
        ],
        out_specs=(
            pl.BlockSpec((_T, _D), lambda l, nb: (nb, 0)),
            pl.BlockSpec((1, _N_TOK, 1), lambda l, nb: (l, 0, 0)),
            scalar_spec, scalar_spec, scalar_spec, scalar_spec,
        ),
        out_shape=out_shapes,
        scratch_shapes=[
            pltpu.VMEM((_N_TOK, _D), jnp.float32),
            pltpu.VMEM((_N_TOK, _D), jnp.float32),
            pltpu.VMEM((1, _K), jnp.float32),
            pltpu.VMEM((1, 1), jnp.float32),
            pltpu.VMEM((1, _K), jnp.float32),
        ],
        compiler_params=pltpu.CompilerParams(
            dimension_semantics=("arbitrary", "arbitrary"),
        ),
    )(zf, cbt, cb_hi, cb_lo, cb_lo2)

    z_q = zq_flat.reshape(z.shape)
    indices = idx3[:, :, 0].T.reshape(z.shape[0], z.shape[1], _NUM_LEVELS)
    return (z_q, indices, loss.reshape(()), cbl.reshape(()),
            coml.reshape(()), perp.reshape(()))


# speedup vs baseline: 1.3223x; 1.1257x over previous
import jax
import jax.numpy as jnp
from jax.experimental import pallas as pl
from jax.experimental.pallas import tpu as pltpu

_NUM_LEVELS = 4
_K = 8192
_D = 256
_BETA = 0.25
_N_TOK = 2304
_T = 256
_NB = _N_TOK // _T


def _rvq_body(z_ref, cb_ref, csq_ref,
              zq_out, idx_out, loss_out, cbl_out, coml_out, perp_out,
              res_s, qsum_s, counts_s, sse_s,
              hi_s, lo_s, lo2_s):
    l = pl.program_id(0)
    nb = pl.program_id(1)
    base = nb * _T

    @pl.when(jnp.logical_and(l == 0, nb == 0))
    def _init_global():
        counts_s[...] = jnp.zeros((1, _K), jnp.float32)
        sse_s[...] = jnp.zeros((1, 1), jnp.float32)

    @pl.when(l == 0)
    def _init_block():
        res_s[pl.ds(base, _T), :] = z_ref[...]
        qsum_s[pl.ds(base, _T), :] = jnp.zeros((_T, _D), jnp.float32)

    res = res_s[pl.ds(base, _T), :]
    cb = cb_ref[0]

    @pl.when(nb == 0)
    def _prep():
        mask = jnp.uint32(0xFFFF0000)
        u0 = jax.lax.bitcast_convert_type(cb, jnp.uint32)
        hi_f = jax.lax.bitcast_convert_type(u0 & mask, jnp.float32)
        r1 = cb - hi_f
        u1 = jax.lax.bitcast_convert_type(r1, jnp.uint32)
        lo_f = jax.lax.bitcast_convert_type(u1 & mask, jnp.float32)
        hi_s[...] = hi_f.astype(jnp.bfloat16)
        lo_s[...] = lo_f.astype(jnp.bfloat16)
        lo2_s[...] = (r1 - lo_f).astype(jnp.bfloat16)

    c_sq = csq_ref[0]
    dot = jax.lax.dot_general(res, cb, (((1,), (1,)), ((), ())))
    d = c_sq - 2.0 * dot
    mval = jnp.min(d, axis=1, keepdims=True)
    lane = jax.lax.broadcasted_iota(jnp.int32, (_T, _K), 1)
    idx = jnp.min(jnp.where(d == mval, lane, _K), axis=1, keepdims=True)

    onehot = (lane == idx).astype(jnp.bfloat16)
    cb_hi = hi_s[...]
    cb_lo = lo_s[...]
    cb_lo2 = lo2_s[...]
    dn = (((1,), (0,)), ((), ()))
    zq = jax.lax.dot_general(onehot, cb_hi, dn,
                             preferred_element_type=jnp.float32)
    zq += jax.lax.dot_general(onehot, cb_lo, dn,
                              preferred_element_type=jnp.float32)
    zq += jax.lax.dot_general(onehot, cb_lo2, dn,
                              preferred_element_type=jnp.float32)

    counts_s[...] += jnp.sum(onehot.astype(jnp.float32), axis=0,
                             keepdims=True)
    sse_s[...] += jnp.sum((zq - res) ** 2, keepdims=True).reshape(1, 1)

    eps = 1e-6
    rn = jnp.sqrt(jnp.sum(res * res, axis=1, keepdims=True))
    u = res / jnp.maximum(rn, eps)
    qn = jnp.sqrt(jnp.sum(zq * zq, axis=1, keepdims=True))
    q = zq / jnp.maximum(qn, eps)
    wv = u + q
    wn = jnp.sqrt(jnp.sum(wv * wv, axis=1, keepdims=True))
    w = wv / jnp.maximum(wn, eps)
    xw = jnp.sum(res * w, axis=1, keepdims=True)
    xu = jnp.sum(res * u, axis=1, keepdims=True)
    rot = res - 2.0 * xw * w + 2.0 * xu * q
    res_s[pl.ds(base, _T), :] = res - rot

    newq = qsum_s[pl.ds(base, _T), :] + zq
    qsum_s[pl.ds(base, _T), :] = newq
    zq_out[...] = newq
    idx_out[0, pl.ds(base, _T), :] = idx

    @pl.when(jnp.logical_and(l == _NUM_LEVELS - 1, nb == _NB - 1))
    def _finalize():
        n_el = jnp.float32(_N_TOK * _D)
        cbl = sse_s[...] / n_el
        probs = counts_s[...] / jnp.float32(_N_TOK * _NUM_LEVELS)
        safe = jnp.where(probs > 0, probs, 1.0)
        ent = -jnp.sum(jnp.where(probs > 0, probs * jnp.log(safe), 0.0),
                       keepdims=True).reshape(1, 1)
        loss_out[...] = cbl * (1.0 + _BETA)
        cbl_out[...] = cbl
        coml_out[...] = cbl
        perp_out[...] = jnp.exp(ent)


def kernel(z, codebooks):
    zf = z.reshape(_N_TOK, _D)
    csq = jnp.sum(codebooks * codebooks, axis=2).reshape(_NUM_LEVELS, 1, _K)

    out_shapes = (
        jax.ShapeDtypeStruct((_N_TOK, _D), jnp.float32),
        jax.ShapeDtypeStruct((_NUM_LEVELS, _N_TOK, 1), jnp.int32),
        jax.ShapeDtypeStruct((1, 1), jnp.float32),
        jax.ShapeDtypeStruct((1, 1), jnp.float32),
        jax.ShapeDtypeStruct((1, 1), jnp.float32),
        jax.ShapeDtypeStruct((1, 1), jnp.float32),
    )
    grid = (_NUM_LEVELS, _NB)
    scalar_spec = pl.BlockSpec((1, 1), lambda l, nb: (0, 0))
    zq_flat, idx3, loss, cbl, coml, perp = pl.pallas_call(
        _rvq_body,
        grid=grid,
        in_specs=[
            pl.BlockSpec((_T, _D), lambda l, nb: (nb, 0)),
            pl.BlockSpec((1, _K, _D), lambda l, nb: (l, 0, 0)),
            pl.BlockSpec((1, 1, _K), lambda l, nb: (l, 0, 0)),
        ],
        out_specs=(
            pl.BlockSpec((_T, _D), lambda l, nb: (nb, 0)),
            pl.BlockSpec((1, _N_TOK, 1), lambda l, nb: (l, 0, 0)),
            scalar_spec, scalar_spec, scalar_spec, scalar_spec,
        ),
        out_shape=out_shapes,
        scratch_shapes=[
            pltpu.VMEM((_N_TOK, _D), jnp.float32),
            pltpu.VMEM((_N_TOK, _D), jnp.float32),
            pltpu.VMEM((1, _K), jnp.float32),
            pltpu.VMEM((1, 1), jnp.float32),
            pltpu.VMEM((_K, _D), jnp.bfloat16),
            pltpu.VMEM((_K, _D), jnp.bfloat16),
            pltpu.VMEM((_K, _D), jnp.bfloat16),
        ],
        compiler_params=pltpu.CompilerParams(
            dimension_semantics=("arbitrary", "arbitrary"),
        ),
    )(zf, codebooks, csq)

    z_q = zq_flat.reshape(z.shape)
    indices = idx3[:, :, 0].T.reshape(z.shape[0], z.shape[1], _NUM_LEVELS)
    return (z_q, indices, loss.reshape(()), cbl.reshape(()),
            coml.reshape(()), perp.reshape(()))


# speedup vs baseline: 1.4848x; 1.1229x over previous
import jax
import jax.numpy as jnp
from jax.experimental import pallas as pl
from jax.experimental.pallas import tpu as pltpu

_NUM_LEVELS = 4
_K = 8192
_D = 256
_BETA = 0.25
_N_TOK = 2304
_T = 256
_NB = _N_TOK // _T


def _rvq_body(z_ref, cb_ref, csq_ref,
              zq_out, idx_out, loss_out, cbl_out, coml_out, perp_out,
              res_s, qsum_s, counts_s, sse_s,
              hi_s, lo_s, lo2_s):
    l = pl.program_id(0)
    nb = pl.program_id(1)
    base = nb * _T

    @pl.when(jnp.logical_and(l == 0, nb == 0))
    def _init_global():
        counts_s[...] = jnp.zeros((1, _K), jnp.float32)
        sse_s[...] = jnp.zeros((1, 1), jnp.float32)

    @pl.when(l == 0)
    def _init_block():
        res_s[pl.ds(base, _T), :] = z_ref[...]
        qsum_s[pl.ds(base, _T), :] = jnp.zeros((_T, _D), jnp.float32)

    res = res_s[pl.ds(base, _T), :]
    cb = cb_ref[0]

    @pl.when(nb == 0)
    def _prep():
        mask = jnp.uint32(0xFFFF0000)
        u0 = jax.lax.bitcast_convert_type(cb, jnp.uint32)
        hi_f = jax.lax.bitcast_convert_type(u0 & mask, jnp.float32)
        r1 = cb - hi_f
        u1 = jax.lax.bitcast_convert_type(r1, jnp.uint32)
        lo_f = jax.lax.bitcast_convert_type(u1 & mask, jnp.float32)
        hi_s[...] = hi_f.astype(jnp.bfloat16)
        lo_s[...] = lo_f.astype(jnp.bfloat16)
        lo2_s[...] = (r1 - lo_f).astype(jnp.bfloat16)

    c_sq = csq_ref[0]
    dot = jax.lax.dot_general(res, cb, (((1,), (1,)), ((), ())))
    d = c_sq - 2.0 * dot
    idx = jnp.argmin(d, axis=1).reshape(_T, 1)
    lane = jax.lax.broadcasted_iota(jnp.int32, (_T, _K), 1)

    onehot = (lane == idx).astype(jnp.bfloat16)
    cb_hi = hi_s[...]
    cb_lo = lo_s[...]
    cb_lo2 = lo2_s[...]
    dn = (((1,), (0,)), ((), ()))
    zq = jax.lax.dot_general(onehot, cb_hi, dn,
                             preferred_element_type=jnp.float32)
    zq += jax.lax.dot_general(onehot, cb_lo, dn,
                              preferred_element_type=jnp.float32)
    zq += jax.lax.dot_general(onehot, cb_lo2, dn,
                              preferred_element_type=jnp.float32)

    counts_s[...] += jnp.sum(onehot.astype(jnp.float32), axis=0,
                             keepdims=True)
    sse_s[...] += jnp.sum((zq - res) ** 2, keepdims=True).reshape(1, 1)

    eps = 1e-6
    rn = jnp.sqrt(jnp.sum(res * res, axis=1, keepdims=True))
    u = res / jnp.maximum(rn, eps)
    qn = jnp.sqrt(jnp.sum(zq * zq, axis=1, keepdims=True))
    q = zq / jnp.maximum(qn, eps)
    wv = u + q
    wn = jnp.sqrt(jnp.sum(wv * wv, axis=1, keepdims=True))
    w = wv / jnp.maximum(wn, eps)
    xw = jnp.sum(res * w, axis=1, keepdims=True)
    xu = jnp.sum(res * u, axis=1, keepdims=True)
    rot = res - 2.0 * xw * w + 2.0 * xu * q
    res_s[pl.ds(base, _T), :] = res - rot

    newq = qsum_s[pl.ds(base, _T), :] + zq
    qsum_s[pl.ds(base, _T), :] = newq
    zq_out[...] = newq
    idx_out[0, pl.ds(base, _T), :] = idx

    @pl.when(jnp.logical_and(l == _NUM_LEVELS - 1, nb == _NB - 1))
    def _finalize():
        n_el = jnp.float32(_N_TOK * _D)
        cbl = sse_s[...] / n_el
        probs = counts_s[...] / jnp.float32(_N_TOK * _NUM_LEVELS)
        safe = jnp.where(probs > 0, probs, 1.0)
        ent = -jnp.sum(jnp.where(probs > 0, probs * jnp.log(safe), 0.0),
                       keepdims=True).reshape(1, 1)
        loss_out[...] = cbl * (1.0 + _BETA)
        cbl_out[...] = cbl
        coml_out[...] = cbl
        perp_out[...] = jnp.exp(ent)


def kernel(z, codebooks):
    zf = z.reshape(_N_TOK, _D)
    csq = jnp.sum(codebooks * codebooks, axis=2).reshape(_NUM_LEVELS, 1, _K)

    out_shapes = (
        jax.ShapeDtypeStruct((_N_TOK, _D), jnp.float32),
        jax.ShapeDtypeStruct((_NUM_LEVELS, _N_TOK, 1), jnp.int32),
        jax.ShapeDtypeStruct((1, 1), jnp.float32),
        jax.ShapeDtypeStruct((1, 1), jnp.float32),
        jax.ShapeDtypeStruct((1, 1), jnp.float32),
        jax.ShapeDtypeStruct((1, 1), jnp.float32),
    )
    grid = (_NUM_LEVELS, _NB)
    scalar_spec = pl.BlockSpec((1, 1), lambda l, nb: (0, 0))
    zq_flat, idx3, loss, cbl, coml, perp = pl.pallas_call(
        _rvq_body,
        grid=grid,
        in_specs=[
            pl.BlockSpec((_T, _D), lambda l, nb: (nb, 0)),
            pl.BlockSpec((1, _K, _D), lambda l, nb: (l, 0, 0)),
            pl.BlockSpec((1, 1, _K), lambda l, nb: (l, 0, 0)),
        ],
        out_specs=(
            pl.BlockSpec((_T, _D), lambda l, nb: (nb, 0)),
            pl.BlockSpec((1, _N_TOK, 1), lambda l, nb: (l, 0, 0)),
            scalar_spec, scalar_spec, scalar_spec, scalar_spec,
        ),
        out_shape=out_shapes,
        scratch_shapes=[
            pltpu.VMEM((_N_TOK, _D), jnp.float32),
            pltpu.VMEM((_N_TOK, _D), jnp.float32),
            pltpu.VMEM((1, _K), jnp.float32),
            pltpu.VMEM((1, 1), jnp.float32),
            pltpu.VMEM((_K, _D), jnp.bfloat16),
            pltpu.VMEM((_K, _D), jnp.bfloat16),
            pltpu.VMEM((_K, _D), jnp.bfloat16),
        ],
        compiler_params=pltpu.CompilerParams(
            dimension_semantics=("arbitrary", "arbitrary"),
        ),
    )(zf, codebooks, csq)

    z_q = zq_flat.reshape(z.shape)
    indices = idx3[:, :, 0].T.reshape(z.shape[0], z.shape[1], _NUM_LEVELS)
    return (z_q, indices, loss.reshape(()), cbl.reshape(()),
            coml.reshape(()), perp.reshape(()))


# speedup vs baseline: 2.3868x; 1.6075x over previous
import functools

import jax
import jax.numpy as jnp
from jax import lax
from jax.experimental import pallas as pl
from jax.experimental.pallas import tpu as pltpu
from jax.experimental.pallas import tpu_sc as plsc

_NUM_LEVELS = 4
_K = 8192
_D = 256
_BETA = 0.25
_N_TOK = 2304
_T = 256
_NB = _N_TOK // _T


def _rotation_residual(res, zq):
    eps = 1e-6
    rn = jnp.sqrt(jnp.sum(res * res, axis=1, keepdims=True))
    u = res / jnp.maximum(rn, eps)
    qn = jnp.sqrt(jnp.sum(zq * zq, axis=1, keepdims=True))
    q = zq / jnp.maximum(qn, eps)
    wv = u + q
    wn = jnp.sqrt(jnp.sum(wv * wv, axis=1, keepdims=True))
    w = wv / jnp.maximum(wn, eps)
    xw = jnp.sum(res * w, axis=1, keepdims=True)
    xu = jnp.sum(res * u, axis=1, keepdims=True)
    rot = res - 2.0 * xw * w + 2.0 * xu * q
    return res - rot


def _argmin_block(res, cb, c_sq):
    dot = lax.dot_general(res, cb, (((1,), (1,)), ((), ())))
    d = c_sq - 2.0 * dot
    return jnp.argmin(d, axis=1).reshape(_T, 1)


def _counts_update(cnt_s, idx, nb):
    @pl.when(nb == 0)
    def _():
        cnt_s[...] = jnp.zeros((1, _K), jnp.float32)

    lane = lax.broadcasted_iota(jnp.int32, (_T, _K), 1)
    cnt_s[...] += jnp.sum((lane == idx).astype(jnp.float32), axis=0,
                          keepdims=True)


def _level0_body(z_ref, cb_ref, csq_ref, idxg_out, cnt_out, cnt_s):
    nb = pl.program_id(0)
    res = z_ref[...]
    idx = _argmin_block(res, cb_ref[0], csq_ref[0])
    idxg_out[...] = idx
    _counts_update(cnt_s, idx, nb)

    @pl.when(nb == _NB - 1)
    def _():
        cnt_out[...] = cnt_s[...]


def _make_tail_level_body(level):
    def body(res_ref, zq_ref, cb_ref, csq_ref,
             idxg_out, res_out, cnt_out, sse_out, cnt_s, sse_s):
        nb = pl.program_id(0)

        @pl.when(nb == 0)
        def _():
            sse_s[...] = jnp.zeros((1, 1), jnp.float32)

        prev = res_ref[...]
        zq = zq_ref[...]
        sse_s[...] += jnp.sum((zq - prev) ** 2,
                              keepdims=True).reshape(1, 1)
        res = _rotation_residual(prev, zq)
        res_out[...] = res
        idx = _argmin_block(res, cb_ref[0], csq_ref[0])
        idxg_out[...] = idx + level * _K
        _counts_update(cnt_s, idx, nb)

        @pl.when(nb == _NB - 1)
        def _():
            cnt_out[...] = cnt_s[...]
            sse_out[...] = sse_s[...]

    return body


def _final_body(res3_ref, zq0_ref, zq1_ref, zq2_ref, zq3_ref,
                c0_ref, c1_ref, c2_ref, c3_ref,
                s0_ref, s1_ref, s2_ref,
                zq_out, loss_out, cbl_out, coml_out, perp_out, sse_s):
    nb = pl.program_id(0)

    @pl.when(nb == 0)
    def _():
        sse_s[...] = jnp.zeros((1, 1), jnp.float32)

    zq3 = zq3_ref[...]
    res3 = res3_ref[...]
    sse_s[...] += jnp.sum((zq3 - res3) ** 2, keepdims=True).reshape(1, 1)
    zq_out[...] = ((zq0_ref[...] + zq1_ref[...]) + zq2_ref[...]) + zq3

    @pl.when(nb == _NB - 1)
    def _():
        n_el = jnp.float32(_N_TOK * _D)
        total = s0_ref[...] + s1_ref[...] + s2_ref[...] + sse_s[...]
        cbl = total / n_el
        counts = ((c0_ref[...] + c1_ref[...]) + c2_ref[...]) + c3_ref[...]
        probs = counts / jnp.float32(_N_TOK * _NUM_LEVELS)
        safe = jnp.where(probs > 0, probs, 1.0)
        ent = -jnp.sum(jnp.where(probs > 0, probs * jnp.log(safe), 0.0),
                       keepdims=True).reshape(1, 1)
        loss_out[...] = cbl * (1.0 + _BETA)
        cbl_out[...] = cbl
        coml_out[...] = cbl
        perp_out[...] = jnp.exp(ent)


_tok_spec = pl.BlockSpec((_T, _D), lambda nb: (nb, 0))
_idx_spec = pl.BlockSpec((_T, 1), lambda nb: (nb, 0))
_cb_spec = lambda level: pl.BlockSpec((1, _K, _D), lambda nb: (level, 0, 0))
_csq_spec = lambda level: pl.BlockSpec((1, 1, _K), lambda nb: (level, 0, 0))
_cnt_spec = pl.BlockSpec((1, _K), lambda nb: (0, 0))
_scal_spec = pl.BlockSpec((1, 1), lambda nb: (0, 0))
_params = pltpu.CompilerParams(dimension_semantics=("arbitrary",))

_IDX_SHAPE = jax.ShapeDtypeStruct((_N_TOK, 1), jnp.int32)
_TOK_SHAPE = jax.ShapeDtypeStruct((_N_TOK, _D), jnp.float32)
_CNT_SHAPE = jax.ShapeDtypeStruct((1, _K), jnp.float32)
_SCAL_SHAPE = jax.ShapeDtypeStruct((1, 1), jnp.float32)


_NC = 2
_NS = 16
_NW = _NC * _NS
_BPW = _N_TOK // _NW
@functools.cache
def _sc_gather_fn():
    mesh = plsc.VectorSubcoreMesh(core_axis_name="c", subcore_axis_name="s")

    @functools.partial(
        pl.kernel, mesh=mesh,
        out_type=jax.ShapeDtypeStruct((_N_TOK, _D), jnp.float32),
        scratch_types=[
            pltpu.VMEM((_BPW,), jnp.int32),
            pltpu.VMEM((_BPW, _D), jnp.float32),
            pltpu.SemaphoreType.DMA,
        ],
    )
    def gather(table_hbm, idx_hbm, out_hbm, idx_v, rows_v, sem):
        wid = lax.axis_index("s") * _NC + lax.axis_index("c")
        base = wid * _BPW
        pltpu.sync_copy(idx_hbm.at[pl.ds(base, _BPW)], idx_v)
        pltpu.async_copy(table_hbm.at[idx_v], rows_v, sem).wait()
        pltpu.sync_copy(rows_v, out_hbm.at[pl.ds(base, _BPW)])

    return gather


def _sc_gather(table, idx):
    return _sc_gather_fn()(table, idx)


def kernel(z, codebooks):
    zf = z.reshape(_N_TOK, _D)
    cb_all = codebooks.reshape(_NUM_LEVELS * _K, _D)
    csq = jnp.sum(codebooks * codebooks, axis=2).reshape(_NUM_LEVELS, 1, _K)

    idxg0, cnt0 = pl.pallas_call(
        _level0_body,
        grid=(_NB,),
        in_specs=[_tok_spec, _cb_spec(0), _csq_spec(0)],
        out_specs=(_idx_spec, _cnt_spec),
        out_shape=(_IDX_SHAPE, _CNT_SHAPE),
        scratch_shapes=[pltpu.VMEM((1, _K), jnp.float32)],
        compiler_params=_params,
    )(zf, codebooks, csq)

    idxgs = [idxg0]
    cnts = [cnt0]
    sses = []
    zqs = []
    res = zf
    ress = []
    for level in range(1, _NUM_LEVELS):
        zq = _sc_gather(cb_all, idxgs[-1].reshape(_N_TOK))
        zqs.append(zq)
        idxg, res, cnt, sse = pl.pallas_call(
            _make_tail_level_body(level),
            grid=(_NB,),
            in_specs=[_tok_spec, _tok_spec, _cb_spec(level),
                      _csq_spec(level)],
            out_specs=(_idx_spec, _tok_spec, _cnt_spec, _scal_spec),
            out_shape=(_IDX_SHAPE, _TOK_SHAPE, _CNT_SHAPE, _SCAL_SHAPE),
            scratch_shapes=[pltpu.VMEM((1, _K), jnp.float32),
                            pltpu.VMEM((1, 1), jnp.float32)],
            compiler_params=_params,
        )(res, zq, codebooks, csq)
        idxgs.append(idxg)
        cnts.append(cnt)
        sses.append(sse)
        ress.append(res)

    zqs.append(_sc_gather(cb_all, idxgs[-1].reshape(_N_TOK)))

    zq_flat, loss, cbl, coml, perp = pl.pallas_call(
        _final_body,
        grid=(_NB,),
        in_specs=[_tok_spec] + [_tok_spec] * 4 + [_cnt_spec] * 4
                 + [_scal_spec] * 3,
        out_specs=(_tok_spec, _scal_spec, _scal_spec, _scal_spec,
                   _scal_spec),
        out_shape=(_TOK_SHAPE, _SCAL_SHAPE, _SCAL_SHAPE, _SCAL_SHAPE,
                   _SCAL_SHAPE),
        scratch_shapes=[pltpu.VMEM((1, 1), jnp.float32)],
        compiler_params=_params,
    )(ress[-1], *zqs, *cnts, *sses)

    z_q = zq_flat.reshape(z.shape)
    offs = jnp.arange(_NUM_LEVELS, dtype=jnp.int32) * _K
    indices = (jnp.concatenate(idxgs, axis=1) - offs[None, :]).reshape(
        z.shape[0], z.shape[1], _NUM_LEVELS)
    return (z_q, indices, loss.reshape(()), cbl.reshape(()),
            coml.reshape(()), perp.reshape(()))


# speedup vs baseline: 2.7219x; 1.1404x over previous
import functools

import jax
import jax.numpy as jnp
from jax import lax
from jax.experimental import pallas as pl
from jax.experimental.pallas import tpu as pltpu
from jax.experimental.pallas import tpu_sc as plsc

_NUM_LEVELS = 4
_K = 8192
_D = 256
_BETA = 0.25
_N_TOK = 2304
_T = 256
_NB = _N_TOK // _T


def _rotation_residual(res, zq):
    eps = 1e-6
    rn = jnp.sqrt(jnp.sum(res * res, axis=1, keepdims=True))
    u = res / jnp.maximum(rn, eps)
    qn = jnp.sqrt(jnp.sum(zq * zq, axis=1, keepdims=True))
    q = zq / jnp.maximum(qn, eps)
    wv = u + q
    wn = jnp.sqrt(jnp.sum(wv * wv, axis=1, keepdims=True))
    w = wv / jnp.maximum(wn, eps)
    xw = jnp.sum(res * w, axis=1, keepdims=True)
    xu = jnp.sum(res * u, axis=1, keepdims=True)
    rot = res - 2.0 * xw * w + 2.0 * xu * q
    return res - rot


def _argmin_block(res, cb, c_sq):
    dot = lax.dot_general(res, cb, (((1,), (1,)), ((), ())))
    d = c_sq - 2.0 * dot
    return jnp.argmin(d, axis=1).reshape(_T, 1)


def _level0_body(z_ref, cb_ref, csq_ref, idxg_out):
    res = z_ref[...]
    idx = _argmin_block(res, cb_ref[0], csq_ref[0])
    idxg_out[...] = idx


def _make_tail_level_body(level):
    def body(res_ref, zq_ref, cb_ref, csq_ref,
             idxg_out, res_out, sse_out, sse_s):
        nb = pl.program_id(0)

        @pl.when(nb == 0)
        def _():
            sse_s[...] = jnp.zeros((1, 1), jnp.float32)

        prev = res_ref[...]
        zq = zq_ref[...]
        sse_s[...] += jnp.sum((zq - prev) ** 2,
                              keepdims=True).reshape(1, 1)
        res = _rotation_residual(prev, zq)
        res_out[...] = res
        idx = _argmin_block(res, cb_ref[0], csq_ref[0])
        idxg_out[...] = idx + level * _K

        @pl.when(nb == _NB - 1)
        def _():
            sse_out[...] = sse_s[...]

    return body


def _final_body(res3_ref, zq0_ref, zq1_ref, zq2_ref, zq3_ref,
                cnt_ref,
                s0_ref, s1_ref, s2_ref,
                zq_out, loss_out, cbl_out, coml_out, perp_out, sse_s):
    nb = pl.program_id(0)

    @pl.when(nb == 0)
    def _():
        sse_s[...] = jnp.zeros((1, 1), jnp.float32)

    zq3 = zq3_ref[...]
    res3 = res3_ref[...]
    sse_s[...] += jnp.sum((zq3 - res3) ** 2, keepdims=True).reshape(1, 1)
    zq_out[...] = ((zq0_ref[...] + zq1_ref[...]) + zq2_ref[...]) + zq3

    @pl.when(nb == _NB - 1)
    def _():
        n_el = jnp.float32(_N_TOK * _D)
        total = s0_ref[...] + s1_ref[...] + s2_ref[...] + sse_s[...]
        cbl = total / n_el
        counts = jnp.sum(cnt_ref[...], axis=0, keepdims=True)
        probs = counts / jnp.float32(_N_TOK * _NUM_LEVELS)
        safe = jnp.where(probs > 0, probs, 1.0)
        ent = -jnp.sum(jnp.where(probs > 0, probs * jnp.log(safe), 0.0),
                       keepdims=True).reshape(1, 1)
        loss_out[...] = cbl * (1.0 + _BETA)
        cbl_out[...] = cbl
        coml_out[...] = cbl
        perp_out[...] = jnp.exp(ent)


_tok_spec = pl.BlockSpec((_T, _D), lambda nb: (nb, 0))
_idx_spec = pl.BlockSpec((_T, 1), lambda nb: (nb, 0))
_cb_spec = lambda level: pl.BlockSpec((1, _K, _D), lambda nb: (level, 0, 0))
_csq_spec = lambda level: pl.BlockSpec((1, 1, _K), lambda nb: (level, 0, 0))
_cnt_spec = pl.BlockSpec((1, _K), lambda nb: (0, 0))
_scal_spec = pl.BlockSpec((1, 1), lambda nb: (0, 0))
_params = pltpu.CompilerParams(dimension_semantics=("arbitrary",))

_IDX_SHAPE = jax.ShapeDtypeStruct((_N_TOK, 1), jnp.int32)
_TOK_SHAPE = jax.ShapeDtypeStruct((_N_TOK, _D), jnp.float32)
_CNT_SHAPE = jax.ShapeDtypeStruct((1, _K), jnp.float32)
_SCAL_SHAPE = jax.ShapeDtypeStruct((1, 1), jnp.float32)


_NC = 2
_NS = 16
_NW = _NC * _NS
_BPW = _N_TOK // _NW
@functools.cache
def _sc_gather_fn():
    mesh = plsc.VectorSubcoreMesh(core_axis_name="c", subcore_axis_name="s")

    @functools.partial(
        pl.kernel, mesh=mesh,
        out_type=jax.ShapeDtypeStruct((_N_TOK, _D), jnp.float32),
        scratch_types=[
            pltpu.VMEM((_BPW,), jnp.int32),
            pltpu.VMEM((_BPW, _D), jnp.float32),
            pltpu.SemaphoreType.DMA,
        ],
    )
    def gather(table_hbm, idx_hbm, out_hbm, idx_v, rows_v, sem):
        wid = lax.axis_index("s") * _NC + lax.axis_index("c")
        base = wid * _BPW
        pltpu.sync_copy(idx_hbm.at[pl.ds(base, _BPW)], idx_v)
        pltpu.async_copy(table_hbm.at[idx_v], rows_v, sem).wait()
        pltpu.sync_copy(rows_v, out_hbm.at[pl.ds(base, _BPW)])

    return gather


def _sc_gather(table, idx):
    return _sc_gather_fn()(table, idx)


_NBINS = _NUM_LEVELS * _K
_SLICE = _NBINS // _NS


@functools.cache
def _sc_bincount_fn():
    mesh = plsc.VectorSubcoreMesh(core_axis_name="c", subcore_axis_name="s")

    @functools.partial(
        pl.kernel, mesh=mesh,
        out_type=jax.ShapeDtypeStruct((_NC, _NBINS), jnp.float32),
        scratch_types=[
            pltpu.VMEM((_NUM_LEVELS, _BPW), jnp.int32),
            pltpu.VMEM((80,), jnp.float32),
            pltpu.VMEM((_SLICE,), jnp.float32),
            pltpu.VMEM_SHARED((_NBINS,), jnp.float32),
            pltpu.SemaphoreType.DMA,
        ],
    )
    def bincount(i0, i1, i2, i3, out_hbm, idx_v, ones_v, zsl_v, hist_sh,
                 sem):
        c = lax.axis_index("c")
        s = lax.axis_index("s")
        wid = s * _NC + c
        base = wid * _BPW
        for j, ih in enumerate((i0, i1, i2, i3)):
            pltpu.sync_copy(ih.at[pl.ds(base, _BPW)], idx_v.at[j])

        def fill_ones(i, _):
            ones_v[pl.ds(i * 16, 16)] = jnp.ones((16,), jnp.float32)
            return 0

        def fill_zero(i, _):
            zsl_v[pl.ds(i * 16, 16)] = jnp.zeros((16,), jnp.float32)
            return 0

        lax.fori_loop(0, 80 // 16, fill_ones, 0)
        lax.fori_loop(0, _SLICE // 16, fill_zero, 0)
        pltpu.sync_copy(zsl_v, hist_sh.at[pl.ds(s * _SLICE, _SLICE)])
        plsc.subcore_barrier()
        for j in range(_NUM_LEVELS):
            pltpu.sync_copy(ones_v.at[pl.ds(0, _BPW)],
                            hist_sh.at[idx_v.at[j]], add=True)
        plsc.subcore_barrier()

        @pl.when(s == 0)
        def _():
            pltpu.sync_copy(hist_sh, out_hbm.at[c])

    return bincount


def _sc_bincount(i0, i1, i2, i3):
    return _sc_bincount_fn()(i0, i1, i2, i3)


def kernel(z, codebooks):
    zf = z.reshape(_N_TOK, _D)
    cb_all = codebooks.reshape(_NUM_LEVELS * _K, _D)
    csq = jnp.sum(codebooks * codebooks, axis=2).reshape(_NUM_LEVELS, 1, _K)

    idxg0 = pl.pallas_call(
        _level0_body,
        grid=(_NB,),
        in_specs=[_tok_spec, _cb_spec(0), _csq_spec(0)],
        out_specs=_idx_spec,
        out_shape=_IDX_SHAPE,
        compiler_params=_params,
    )(zf, codebooks, csq)

    idxgs = [idxg0]
    sses = []
    zqs = []
    res = zf
    ress = []
    for level in range(1, _NUM_LEVELS):
        zq = _sc_gather(cb_all, idxgs[-1].reshape(_N_TOK))
        zqs.append(zq)
        idxg, res, sse = pl.pallas_call(
            _make_tail_level_body(level),
            grid=(_NB,),
            in_specs=[_tok_spec, _tok_spec, _cb_spec(level),
                      _csq_spec(level)],
            out_specs=(_idx_spec, _tok_spec, _scal_spec),
            out_shape=(_IDX_SHAPE, _TOK_SHAPE, _SCAL_SHAPE),
            scratch_shapes=[pltpu.VMEM((1, 1), jnp.float32)],
            compiler_params=_params,
        )(res, zq, codebooks, csq)
        idxgs.append(idxg)
        sses.append(sse)
        ress.append(res)

    zqs.append(_sc_gather(cb_all, idxgs[-1].reshape(_N_TOK)))
    hist2 = _sc_bincount(*[ig.reshape(_N_TOK) for ig in idxgs])
    hist = hist2.reshape(_NC * _NUM_LEVELS, _K)

    zq_flat, loss, cbl, coml, perp = pl.pallas_call(
        _final_body,
        grid=(_NB,),
        in_specs=[_tok_spec] + [_tok_spec] * 4
                 + [pl.BlockSpec((_NC * _NUM_LEVELS, _K),
                                 lambda nb: (0, 0))]
                 + [_scal_spec] * 3,
        out_specs=(_tok_spec, _scal_spec, _scal_spec, _scal_spec,
                   _scal_spec),
        out_shape=(_TOK_SHAPE, _SCAL_SHAPE, _SCAL_SHAPE, _SCAL_SHAPE,
                   _SCAL_SHAPE),
        scratch_shapes=[pltpu.VMEM((1, 1), jnp.float32)],
        compiler_params=_params,
    )(ress[-1], *zqs, hist, *sses)

    z_q = zq_flat.reshape(z.shape)
    offs = jnp.arange(_NUM_LEVELS, dtype=jnp.int32) * _K
    indices = (jnp.concatenate(idxgs, axis=1) - offs[None, :]).reshape(
        z.shape[0], z.shape[1], _NUM_LEVELS)
    return (z_q, indices, loss.reshape(()), cbl.reshape(()),
            coml.reshape(()), perp.reshape(()))


# speedup vs baseline: 2.7255x; 1.0013x over previous
import functools

import jax
import jax.numpy as jnp
from jax import lax
from jax.experimental import pallas as pl
from jax.experimental.pallas import tpu as pltpu
from jax.experimental.pallas import tpu_sc as plsc

_NUM_LEVELS = 4
_K = 8192
_D = 256
_BETA = 0.25
_N_TOK = 2304
_T = 256
_NB = _N_TOK // _T


def _rotation_residual(res, zq):
    eps = 1e-6
    rn = jnp.sqrt(jnp.sum(res * res, axis=1, keepdims=True))
    u = res / jnp.maximum(rn, eps)
    qn = jnp.sqrt(jnp.sum(zq * zq, axis=1, keepdims=True))
    q = zq / jnp.maximum(qn, eps)
    wv = u + q
    wn = jnp.sqrt(jnp.sum(wv * wv, axis=1, keepdims=True))
    w = wv / jnp.maximum(wn, eps)
    xw = jnp.sum(res * w, axis=1, keepdims=True)
    xu = jnp.sum(res * u, axis=1, keepdims=True)
    rot = res - 2.0 * xw * w + 2.0 * xu * q
    return res - rot


def _argmin_block(res, cb, c_sq):
    dot = lax.dot_general(res, cb, (((1,), (1,)), ((), ())))
    d = c_sq - 2.0 * dot
    return jnp.argmin(d, axis=1).reshape(_T, 1)


def _level0_body(z_ref, cb_ref, csq_ref, idxg_out):
    res = z_ref[...]
    idx = _argmin_block(res, cb_ref[0], csq_ref[0])
    idxg_out[...] = idx


def _make_tail_level_body(level):
    def body(res_ref, zq_ref, cb_ref, csq_ref,
             idxg_out, res_out, sse_out, sse_s):
        nb = pl.program_id(0)

        @pl.when(nb == 0)
        def _():
            sse_s[...] = jnp.zeros((1, 1), jnp.float32)

        prev = res_ref[...]
        zq = zq_ref[...]
        sse_s[...] += jnp.sum((zq - prev) ** 2,
                              keepdims=True).reshape(1, 1)
        res = _rotation_residual(prev, zq)
        res_out[...] = res
        idx = _argmin_block(res, cb_ref[0], csq_ref[0])
        idxg_out[...] = idx + level * _K

        @pl.when(nb == _NB - 1)
        def _():
            sse_out[...] = sse_s[...]

    return body


def _final_body(res3_ref, zq0_ref, zq1_ref, zq2_ref, zq3_ref,
                cnt_ref,
                s0_ref, s1_ref, s2_ref,
                zq_out, loss_out, cbl_out, coml_out, perp_out, sse_s):
    nb = pl.program_id(0)

    @pl.when(nb == 0)
    def _():
        sse_s[...] = jnp.zeros((1, 1), jnp.float32)

    zq3 = zq3_ref[...]
    res3 = res3_ref[...]
    sse_s[...] += jnp.sum((zq3 - res3) ** 2, keepdims=True).reshape(1, 1)
    zq_out[...] = ((zq0_ref[...] + zq1_ref[...]) + zq2_ref[...]) + zq3

    @pl.when(nb == _NB - 1)
    def _():
        n_el = jnp.float32(_N_TOK * _D)
        total = s0_ref[...] + s1_ref[...] + s2_ref[...] + sse_s[...]
        cbl = total / n_el
        counts = jnp.sum(cnt_ref[...], axis=0, keepdims=True)
        probs = counts / jnp.float32(_N_TOK * _NUM_LEVELS)
        safe = jnp.where(probs > 0, probs, 1.0)
        ent = -jnp.sum(jnp.where(probs > 0, probs * jnp.log(safe), 0.0),
                       keepdims=True).reshape(1, 1)
        loss_out[...] = cbl * (1.0 + _BETA)
        cbl_out[...] = cbl
        coml_out[...] = cbl
        perp_out[...] = jnp.exp(ent)


_tok_spec = pl.BlockSpec((_T, _D), lambda nb: (nb, 0))
_idx_spec = pl.BlockSpec((_T, 1), lambda nb: (nb, 0))
_cb_spec = lambda level: pl.BlockSpec((1, _K, _D), lambda nb: (level, 0, 0))
_csq_spec = lambda level: pl.BlockSpec((1, 1, _K), lambda nb: (level, 0, 0))
_scal_spec = pl.BlockSpec((1, 1), lambda nb: (0, 0))
_params = pltpu.CompilerParams(dimension_semantics=("arbitrary",))

_IDX_SHAPE = jax.ShapeDtypeStruct((_N_TOK, 1), jnp.int32)
_TOK_SHAPE = jax.ShapeDtypeStruct((_N_TOK, _D), jnp.float32)
_SCAL_SHAPE = jax.ShapeDtypeStruct((1, 1), jnp.float32)


_NC = 2
_NS = 16
_NW = _NC * _NS
_BPW = _N_TOK // _NW
@functools.cache
def _sc_gather_fn():
    mesh = plsc.VectorSubcoreMesh(core_axis_name="c", subcore_axis_name="s")

    @functools.partial(
        pl.kernel, mesh=mesh,
        out_type=jax.ShapeDtypeStruct((_N_TOK, _D), jnp.float32),
        scratch_types=[
            pltpu.VMEM((_BPW,), jnp.int32),
            pltpu.VMEM((_BPW, _D), jnp.float32),
            pltpu.SemaphoreType.DMA,
        ],
    )
    def gather(table_hbm, idx_hbm, out_hbm, idx_v, rows_v, sem):
        wid = lax.axis_index("s") * _NC + lax.axis_index("c")
        base = wid * _BPW
        pltpu.sync_copy(idx_hbm.at[pl.ds(base, _BPW)], idx_v)
        pltpu.async_copy(table_hbm.at[idx_v], rows_v, sem).wait()
        pltpu.sync_copy(rows_v, out_hbm.at[pl.ds(base, _BPW)])

    return gather


def _sc_gather(table, idx):
    return _sc_gather_fn()(table, idx)


_NBINS = _NUM_LEVELS * _K
_SLICE = _NBINS // _NS


@functools.cache
def _sc_bincount_fn():
    mesh = plsc.VectorSubcoreMesh(core_axis_name="c", subcore_axis_name="s")

    @functools.partial(
        pl.kernel, mesh=mesh,
        out_type=jax.ShapeDtypeStruct((_NC, _NBINS), jnp.float32),
        scratch_types=[
            pltpu.VMEM((_NUM_LEVELS, _BPW), jnp.int32),
            pltpu.VMEM((80,), jnp.float32),
            pltpu.VMEM((_SLICE,), jnp.float32),
            pltpu.VMEM_SHARED((_NBINS,), jnp.float32),
            pltpu.SemaphoreType.DMA,
        ],
    )
    def bincount(i0, i1, i2, i3, out_hbm, idx_v, ones_v, zsl_v, hist_sh,
                 sem):
        c = lax.axis_index("c")
        s = lax.axis_index("s")
        wid = s * _NC + c
        base = wid * _BPW
        for j, ih in enumerate((i0, i1, i2, i3)):
            pltpu.sync_copy(ih.at[pl.ds(base, _BPW)], idx_v.at[j])

        def fill_ones(i, _):
            ones_v[pl.ds(i * 16, 16)] = jnp.ones((16,), jnp.float32)
            return 0

        def fill_zero(i, _):
            zsl_v[pl.ds(i * 16, 16)] = jnp.zeros((16,), jnp.float32)
            return 0

        lax.fori_loop(0, 80 // 16, fill_ones, 0)
        lax.fori_loop(0, _SLICE // 16, fill_zero, 0)
        pltpu.sync_copy(zsl_v, hist_sh.at[pl.ds(s * _SLICE, _SLICE)])
        plsc.subcore_barrier()
        for j in range(_NUM_LEVELS):
            pltpu.sync_copy(ones_v.at[pl.ds(0, _BPW)],
                            hist_sh.at[idx_v.at[j]], add=True)
        plsc.subcore_barrier()

        @pl.when(s == 0)
        def _():
            pltpu.sync_copy(hist_sh, out_hbm.at[c])

    return bincount


def _sc_bincount(i0, i1, i2, i3):
    return _sc_bincount_fn()(i0, i1, i2, i3)


def kernel(z, codebooks):
    zf = z.reshape(_N_TOK, _D)
    cb_all = codebooks.reshape(_NUM_LEVELS * _K, _D)
    csq = jnp.sum(codebooks * codebooks, axis=2).reshape(_NUM_LEVELS, 1, _K)

    idxg0 = pl.pallas_call(
        _level0_body,
        grid=(_NB,),
        in_specs=[_tok_spec, _cb_spec(0), _csq_spec(0)],
        out_specs=_idx_spec,
        out_shape=_IDX_SHAPE,
        compiler_params=_params,
    )(zf, codebooks, csq)

    idxgs = [idxg0]
    sses = []
    zqs = []
    res = zf
    ress = []
    for level in range(1, _NUM_LEVELS):
        zq = _sc_gather(cb_all, idxgs[-1].reshape(_N_TOK))
        zqs.append(zq)
        idxg, res, sse = pl.pallas_call(
            _make_tail_level_body(level),
            grid=(_NB,),
            in_specs=[_tok_spec, _tok_spec, _cb_spec(level),
                      _csq_spec(level)],
            out_specs=(_idx_spec, _tok_spec, _scal_spec),
            out_shape=(_IDX_SHAPE, _TOK_SHAPE, _SCAL_SHAPE),
            scratch_shapes=[pltpu.VMEM((1, 1), jnp.float32)],
            compiler_params=_params,
        )(res, zq, codebooks, csq)
        idxgs.append(idxg)
        sses.append(sse)
        ress.append(res)

    zqs.append(_sc_gather(cb_all, idxgs[-1].reshape(_N_TOK)))
    hist2 = _sc_bincount(*[ig.reshape(_N_TOK) for ig in idxgs])
    hist = hist2.reshape(_NC * _NUM_LEVELS, _K)

    zq_flat, loss, cbl, coml, perp = pl.pallas_call(
        _final_body,
        grid=(_NB,),
        in_specs=[_tok_spec] + [_tok_spec] * 4
                 + [pl.BlockSpec((_NC * _NUM_LEVELS, _K),
                                 lambda nb: (0, 0))]
                 + [_scal_spec] * 3,
        out_specs=(_tok_spec, _scal_spec, _scal_spec, _scal_spec,
                   _scal_spec),
        out_shape=(_TOK_SHAPE, _SCAL_SHAPE, _SCAL_SHAPE, _SCAL_SHAPE,
                   _SCAL_SHAPE),
        scratch_shapes=[pltpu.VMEM((1, 1), jnp.float32)],
        compiler_params=_params,
    )(ress[-1], *zqs, hist, *sses)

    z_q = zq_flat.reshape(z.shape)
    offs = jnp.arange(_NUM_LEVELS, dtype=jnp.int32) * _K
    indices = (jnp.concatenate(idxgs, axis=1) - offs[None, :]).reshape(
        z.shape[0], z.shape[1], _NUM_LEVELS)
    return (z_q, indices, loss.reshape(()), cbl.reshape(()),
            coml.reshape(()), perp.reshape(()))


# speedup vs baseline: 2.8913x; 1.0608x over previous
import functools

import jax
import jax.numpy as jnp
from jax import lax
from jax.experimental import pallas as pl
from jax.experimental.pallas import tpu as pltpu
from jax.experimental.pallas import tpu_sc as plsc

_NUM_LEVELS = 4
_K = 8192
_D = 256
_BETA = 0.25
_N_TOK = 2304
_T = 576
_NB = _N_TOK // _T


def _rotation_residual(res, zq):
    eps = 1e-6
    rn = jnp.sqrt(jnp.sum(res * res, axis=1, keepdims=True))
    u = res / jnp.maximum(rn, eps)
    qn = jnp.sqrt(jnp.sum(zq * zq, axis=1, keepdims=True))
    q = zq / jnp.maximum(qn, eps)
    wv = u + q
    wn = jnp.sqrt(jnp.sum(wv * wv, axis=1, keepdims=True))
    w = wv / jnp.maximum(wn, eps)
    xw = jnp.sum(res * w, axis=1, keepdims=True)
    xu = jnp.sum(res * u, axis=1, keepdims=True)
    rot = res - 2.0 * xw * w + 2.0 * xu * q
    return res - rot


def _argmin_block(res, cb, c_sq):
    dot = lax.dot_general(res, cb, (((1,), (1,)), ((), ())))
    d = c_sq - 2.0 * dot
    return jnp.argmin(d, axis=1).reshape(_T, 1)


def _level0_body(z_ref, cb_ref, csq_ref, idxg_out):
    res = z_ref[...]
    idx = _argmin_block(res, cb_ref[0], csq_ref[0])
    idxg_out[...] = idx


def _make_tail_level_body(level):
    def body(res_ref, zq_ref, cb_ref, csq_ref,
             idxg_out, res_out, sse_out, sse_s):
        nb = pl.program_id(0)

        @pl.when(nb == 0)
        def _():
            sse_s[...] = jnp.zeros((1, 1), jnp.float32)

        prev = res_ref[...]
        zq = zq_ref[...]
        sse_s[...] += jnp.sum((zq - prev) ** 2,
                              keepdims=True).reshape(1, 1)
        res = _rotation_residual(prev, zq)
        res_out[...] = res
        idx = _argmin_block(res, cb_ref[0], csq_ref[0])
        idxg_out[...] = idx + level * _K

        @pl.when(nb == _NB - 1)
        def _():
            sse_out[...] = sse_s[...]

    return body


def _final_body(res3_ref, zq0_ref, zq1_ref, zq2_ref, zq3_ref,
                cnt_ref,
                s0_ref, s1_ref, s2_ref,
                zq_out, loss_out, cbl_out, coml_out, perp_out, sse_s):
    nb = pl.program_id(0)

    @pl.when(nb == 0)
    def _():
        sse_s[...] = jnp.zeros((1, 1), jnp.float32)

    zq3 = zq3_ref[...]
    res3 = res3_ref[...]
    sse_s[...] += jnp.sum((zq3 - res3) ** 2, keepdims=True).reshape(1, 1)
    zq_out[...] = ((zq0_ref[...] + zq1_ref[...]) + zq2_ref[...]) + zq3

    @pl.when(nb == _NB - 1)
    def _():
        n_el = jnp.float32(_N_TOK * _D)
        total = s0_ref[...] + s1_ref[...] + s2_ref[...] + sse_s[...]
        cbl = total / n_el
        counts = jnp.sum(cnt_ref[...], axis=0, keepdims=True)
        probs = counts / jnp.float32(_N_TOK * _NUM_LEVELS)
        safe = jnp.where(probs > 0, probs, 1.0)
        ent = -jnp.sum(jnp.where(probs > 0, probs * jnp.log(safe), 0.0),
                       keepdims=True).reshape(1, 1)
        loss_out[...] = cbl * (1.0 + _BETA)
        cbl_out[...] = cbl
        coml_out[...] = cbl
        perp_out[...] = jnp.exp(ent)


_tok_spec = pl.BlockSpec((_T, _D), lambda nb: (nb, 0))
_idx_spec = pl.BlockSpec((_T, 1), lambda nb: (nb, 0))
_cb_spec = lambda level: pl.BlockSpec((1, _K, _D), lambda nb: (level, 0, 0))
_csq_spec = lambda level: pl.BlockSpec((1, 1, _K), lambda nb: (level, 0, 0))
_scal_spec = pl.BlockSpec((1, 1), lambda nb: (0, 0))
_params = pltpu.CompilerParams(dimension_semantics=("arbitrary",))

_IDX_SHAPE = jax.ShapeDtypeStruct((_N_TOK, 1), jnp.int32)
_TOK_SHAPE = jax.ShapeDtypeStruct((_N_TOK, _D), jnp.float32)
_SCAL_SHAPE = jax.ShapeDtypeStruct((1, 1), jnp.float32)


_NC = 2
_NS = 16
_NW = _NC * _NS
_BPW = _N_TOK // _NW
@functools.cache
def _sc_gather_fn():
    mesh = plsc.VectorSubcoreMesh(core_axis_name="c", subcore_axis_name="s")

    @functools.partial(
        pl.kernel, mesh=mesh,
        out_type=jax.ShapeDtypeStruct((_N_TOK, _D), jnp.float32),
        scratch_types=[
            pltpu.VMEM((_BPW,), jnp.int32),
            pltpu.VMEM((_BPW, _D), jnp.float32),
            pltpu.SemaphoreType.DMA,
        ],
    )
    def gather(table_hbm, idx_hbm, out_hbm, idx_v, rows_v, sem):
        wid = lax.axis_index("s") * _NC + lax.axis_index("c")
        base = wid * _BPW
        pltpu.sync_copy(idx_hbm.at[pl.ds(base, _BPW)], idx_v)
        pltpu.async_copy(table_hbm.at[idx_v], rows_v, sem).wait()
        pltpu.sync_copy(rows_v, out_hbm.at[pl.ds(base, _BPW)])

    return gather


def _sc_gather(table, idx):
    return _sc_gather_fn()(table, idx)


_NBINS = _NUM_LEVELS * _K
_SLICE = _NBINS // _NS


@functools.cache
def _sc_bincount_fn():
    mesh = plsc.VectorSubcoreMesh(core_axis_name="c", subcore_axis_name="s")

    @functools.partial(
        pl.kernel, mesh=mesh,
        out_type=jax.ShapeDtypeStruct((_NC, _NBINS), jnp.float32),
        scratch_types=[
            pltpu.VMEM((_NUM_LEVELS, _BPW), jnp.int32),
            pltpu.VMEM((80,), jnp.float32),
            pltpu.VMEM((_SLICE,), jnp.float32),
            pltpu.VMEM_SHARED((_NBINS,), jnp.float32),
            pltpu.SemaphoreType.DMA,
        ],
    )
    def bincount(i0, i1, i2, i3, out_hbm, idx_v, ones_v, zsl_v, hist_sh,
                 sem):
        c = lax.axis_index("c")
        s = lax.axis_index("s")
        wid = s * _NC + c
        base = wid * _BPW
        for j, ih in enumerate((i0, i1, i2, i3)):
            pltpu.sync_copy(ih.at[pl.ds(base, _BPW)], idx_v.at[j])

        def fill_ones(i, _):
            ones_v[pl.ds(i * 16, 16)] = jnp.ones((16,), jnp.float32)
            return 0

        def fill_zero(i, _):
            zsl_v[pl.ds(i * 16, 16)] = jnp.zeros((16,), jnp.float32)
            return 0

        lax.fori_loop(0, 80 // 16, fill_ones, 0)
        lax.fori_loop(0, _SLICE // 16, fill_zero, 0)
        pltpu.sync_copy(zsl_v, hist_sh.at[pl.ds(s * _SLICE, _SLICE)])
        plsc.subcore_barrier()
        for j in range(_NUM_LEVELS):
            pltpu.sync_copy(ones_v.at[pl.ds(0, _BPW)],
                            hist_sh.at[idx_v.at[j]], add=True)
        plsc.subcore_barrier()

        @pl.when(s == 0)
        def _():
            pltpu.sync_copy(hist_sh, out_hbm.at[c])

    return bincount


def _sc_bincount(i0, i1, i2, i3):
    return _sc_bincount_fn()(i0, i1, i2, i3)


def kernel(z, codebooks):
    zf = z.reshape(_N_TOK, _D)
    cb_all = codebooks.reshape(_NUM_LEVELS * _K, _D)
    csq = jnp.sum(codebooks * codebooks, axis=2).reshape(_NUM_LEVELS, 1, _K)

    idxg0 = pl.pallas_call(
        _level0_body,
        grid=(_NB,),
        in_specs=[_tok_spec, _cb_spec(0), _csq_spec(0)],
        out_specs=_idx_spec,
        out_shape=_IDX_SHAPE,
        compiler_params=_params,
    )(zf, codebooks, csq)

    idxgs = [idxg0]
    sses = []
    zqs = []
    res = zf
    ress = []
    for level in range(1, _NUM_LEVELS):
        zq = _sc_gather(cb_all, idxgs[-1].reshape(_N_TOK))
        zqs.append(zq)
        idxg, res, sse = pl.pallas_call(
            _make_tail_level_body(level),
            grid=(_NB,),
            in_specs=[_tok_spec, _tok_spec, _cb_spec(level),
                      _csq_spec(level)],
            out_specs=(_idx_spec, _tok_spec, _scal_spec),
            out_shape=(_IDX_SHAPE, _TOK_SHAPE, _SCAL_SHAPE),
            scratch_shapes=[pltpu.VMEM((1, 1), jnp.float32)],
            compiler_params=_params,
        )(res, zq, codebooks, csq)
        idxgs.append(idxg)
        sses.append(sse)
        ress.append(res)

    zqs.append(_sc_gather(cb_all, idxgs[-1].reshape(_N_TOK)))
    hist2 = _sc_bincount(*[ig.reshape(_N_TOK) for ig in idxgs])
    hist = hist2.reshape(_NC * _NUM_LEVELS, _K)

    zq_flat, loss, cbl, coml, perp = pl.pallas_call(
        _final_body,
        grid=(_NB,),
        in_specs=[_tok_spec] + [_tok_spec] * 4
                 + [pl.BlockSpec((_NC * _NUM_LEVELS, _K),
                                 lambda nb: (0, 0))]
                 + [_scal_spec] * 3,
        out_specs=(_tok_spec, _scal_spec, _scal_spec, _scal_spec,
                   _scal_spec),
        out_shape=(_TOK_SHAPE, _SCAL_SHAPE, _SCAL_SHAPE, _SCAL_SHAPE,
                   _SCAL_SHAPE),
        scratch_shapes=[pltpu.VMEM((1, 1), jnp.float32)],
        compiler_params=_params,
    )(ress[-1], *zqs, hist, *sses)

    z_q = zq_flat.reshape(z.shape)
    offs = jnp.arange(_NUM_LEVELS, dtype=jnp.int32) * _K
    indices = (jnp.concatenate(idxgs, axis=1) - offs[None, :]).reshape(
        z.shape[0], z.shape[1], _NUM_LEVELS)
    return (z_q, indices, loss.reshape(()), cbl.reshape(()),
            coml.reshape(()), perp.reshape(()))
